# Initial kernel scaffold; baseline (speedup 1.0000x reference)
import jax
import jax.numpy as jnp
from jax.experimental import pallas as pl


def kernel(x, Wg, bg, Wnoise, bnoise, eps):
    raise NotImplementedError("write your pallas kernel here")


# speedup vs baseline: 4.4716x; 4.4716x over previous
import functools

import jax
import jax.numpy as jnp
from jax import lax
from jax.experimental import pallas as pl

B, S, D, E = 4, 8192, 768, 8
BLK = 2048


def _body(x_ref, wg_ref, bg_ref, wn_ref, bn_ref, eps_ref, out_ref):
    xb = x_ref[...]
    dn = (((1,), (1,)), ((), ()))
    gate = lax.dot_general(xb, wg_ref[...], dn,
                           preferred_element_type=jnp.float32) + bg_ref[...]
    noise = lax.dot_general(xb, wn_ref[...], dn,
                            preferred_element_type=jnp.float32) + bn_ref[...]
    h = gate + eps_ref[...] * jax.nn.softplus(noise)

    e_iota = lax.broadcasted_iota(jnp.int32, (BLK, E), 1)
    m1 = jnp.max(h, axis=1, keepdims=True)
    i1 = jnp.min(jnp.where(h == m1, e_iota, E), axis=1, keepdims=True)
    mask1 = e_iota == i1
    h2 = jnp.where(mask1, -jnp.inf, h)
    m2 = jnp.max(h2, axis=1, keepdims=True)
    i2 = jnp.min(jnp.where(h2 == m2, e_iota, E), axis=1, keepdims=True)
    mask2 = e_iota == i2
    e2 = jnp.exp(m2 - m1)
    inv_denom = 1.0 / (1.0 + e2)
    out_ref[...] = jnp.where(mask1, inv_denom,
                             jnp.where(mask2, e2 * inv_denom, 0.0))


@jax.jit
def kernel(x, Wg, bg, Wnoise, bnoise, eps):
    n_tok = B * S
    x2 = x.reshape(n_tok, D)
    eps2 = eps.reshape(n_tok, E)
    grid = (n_tok // BLK,)
    out = pl.pallas_call(
        _body,
        grid=grid,
        in_specs=[
            pl.BlockSpec((BLK, D), lambda i: (i, 0)),
            pl.BlockSpec((E, D), lambda i: (0, 0)),
            pl.BlockSpec((1, E), lambda i: (0, 0)),
            pl.BlockSpec((E, D), lambda i: (0, 0)),
            pl.BlockSpec((1, E), lambda i: (0, 0)),
            pl.BlockSpec((BLK, E), lambda i: (i, 0)),
        ],
        out_specs=pl.BlockSpec((BLK, E), lambda i: (i, 0)),
        out_shape=jax.ShapeDtypeStruct((n_tok, E), jnp.float32),
    )(x2, Wg, bg.reshape(1, E), Wnoise, bnoise.reshape(1, E), eps2)
    return out.reshape(B, S, E)


# speedup vs baseline: 5.1708x; 1.1564x over previous
import functools

import jax
import jax.numpy as jnp
from jax import lax
from jax.experimental import pallas as pl

B, S, D, E = 4, 8192, 768, 8
BLK = 2048


def _body(x_ref, wg_ref, bg_ref, wn_ref, bn_ref, eps_ref, out_ref):
    xb = x_ref[...]
    dn = (((1,), (1,)), ((), ()))
    gate = lax.dot_general(wg_ref[...], xb, dn,
                           preferred_element_type=jnp.float32) + bg_ref[...]
    noise = lax.dot_general(wn_ref[...], xb, dn,
                            preferred_element_type=jnp.float32) + bn_ref[...]
    h = gate + eps_ref[...].T * jax.nn.softplus(noise)

    e_iota = lax.broadcasted_iota(jnp.int32, (E, BLK), 0)
    m1 = jnp.max(h, axis=0, keepdims=True)
    i1 = jnp.min(jnp.where(h == m1, e_iota, E), axis=0, keepdims=True)
    mask1 = e_iota == i1
    h2 = jnp.where(mask1, -jnp.inf, h)
    m2 = jnp.max(h2, axis=0, keepdims=True)
    i2 = jnp.min(jnp.where(h2 == m2, e_iota, E), axis=0, keepdims=True)
    mask2 = e_iota == i2
    e2 = jnp.exp(m2 - m1)
    inv_denom = 1.0 / (1.0 + e2)
    out_ref[...] = jnp.where(mask1, inv_denom,
                             jnp.where(mask2, e2 * inv_denom, 0.0)).T


@jax.jit
def kernel(x, Wg, bg, Wnoise, bnoise, eps):
    n_tok = B * S
    x2 = x.reshape(n_tok, D)
    eps2 = eps.reshape(n_tok, E)
    grid = (n_tok // BLK,)
    out = pl.pallas_call(
        _body,
        grid=grid,
        in_specs=[
            pl.BlockSpec((BLK, D), lambda i: (i, 0)),
            pl.BlockSpec((E, D), lambda i: (0, 0)),
            pl.BlockSpec((E, 1), lambda i: (0, 0)),
            pl.BlockSpec((E, D), lambda i: (0, 0)),
            pl.BlockSpec((E, 1), lambda i: (0, 0)),
            pl.BlockSpec((BLK, E), lambda i: (i, 0)),
        ],
        out_specs=pl.BlockSpec((BLK, E), lambda i: (i, 0)),
        out_shape=jax.ShapeDtypeStruct((n_tok, E), jnp.float32),
    )(x2, Wg, bg.reshape(E, 1), Wnoise, bnoise.reshape(E, 1), eps2)
    return out.reshape(B, S, E)


# speedup vs baseline: 5.3593x; 1.0364x over previous
import functools

import jax
import jax.numpy as jnp
from jax import lax
from jax.experimental import pallas as pl

B, S, D, E = 4, 8192, 768, 8
BLK = 4096


def _body(x_ref, wg_ref, bg_ref, wn_ref, bn_ref, eps_ref, out_ref):
    xb = x_ref[...]
    dn = (((1,), (1,)), ((), ()))
    gate = lax.dot_general(wg_ref[...], xb, dn,
                           preferred_element_type=jnp.float32) + bg_ref[...]
    noise = lax.dot_general(wn_ref[...], xb, dn,
                            preferred_element_type=jnp.float32) + bn_ref[...]
    h = gate + eps_ref[...].T * jax.nn.softplus(noise)

    e_iota = lax.broadcasted_iota(jnp.int32, (E, BLK), 0)
    m1 = jnp.max(h, axis=0, keepdims=True)
    i1 = jnp.min(jnp.where(h == m1, e_iota, E), axis=0, keepdims=True)
    mask1 = e_iota == i1
    h2 = jnp.where(mask1, -jnp.inf, h)
    m2 = jnp.max(h2, axis=0, keepdims=True)
    i2 = jnp.min(jnp.where(h2 == m2, e_iota, E), axis=0, keepdims=True)
    mask2 = e_iota == i2
    e2 = jnp.exp(m2 - m1)
    inv_denom = 1.0 / (1.0 + e2)
    out_ref[...] = jnp.where(mask1, inv_denom,
                             jnp.where(mask2, e2 * inv_denom, 0.0)).T


@jax.jit
def kernel(x, Wg, bg, Wnoise, bnoise, eps):
    n_tok = B * S
    x2 = x.reshape(n_tok, D)
    eps2 = eps.reshape(n_tok, E)
    grid = (n_tok // BLK,)
    out = pl.pallas_call(
        _body,
        grid=grid,
        in_specs=[
            pl.BlockSpec((BLK, D), lambda i: (i, 0)),
            pl.BlockSpec((E, D), lambda i: (0, 0)),
            pl.BlockSpec((E, 1), lambda i: (0, 0)),
            pl.BlockSpec((E, D), lambda i: (0, 0)),
            pl.BlockSpec((E, 1), lambda i: (0, 0)),
            pl.BlockSpec((BLK, E), lambda i: (i, 0)),
        ],
        out_specs=pl.BlockSpec((BLK, E), lambda i: (i, 0)),
        out_shape=jax.ShapeDtypeStruct((n_tok, E), jnp.float32),
    )(x2, Wg, bg.reshape(E, 1), Wnoise, bnoise.reshape(E, 1), eps2)
    return out.reshape(B, S, E)
